# baseline (speedup 1.0000x reference)
import functools

import jax
import jax.numpy as jnp
from jax import lax
from jax.experimental import pallas as pl
from jax.experimental.pallas import tpu as pltpu
from jax.experimental.pallas import tpu_sc as plsc

_NC = 2
_NS = 16
_NW = _NC * _NS

_TILE_V = 2048


def _sc_gather(emb, idx):
    B = idx.shape[0]
    D = emb.shape[1]
    bpw = B // _NW
    mesh = plsc.VectorSubcoreMesh(
        core_axis_name="c", subcore_axis_name="s",
        num_cores=_NC, num_subcores=_NS,
    )

    @functools.partial(
        pl.kernel,
        out_type=jax.ShapeDtypeStruct((B, D), jnp.float32),
        mesh=mesh,
        compiler_params=pltpu.CompilerParams(use_tc_tiling_on_sc=False),
        scratch_types=[
            pltpu.VMEM((bpw,), jnp.int32),
            pltpu.VMEM((bpw, D), jnp.float32),
            pltpu.SemaphoreType.DMA,
        ],
    )
    def gather_kernel(emb_hbm, idx_hbm, out_hbm, idx_v, rows_v, sem):
        wid = lax.axis_index("s") * _NC + lax.axis_index("c")
        base = wid * bpw
        pltpu.sync_copy(idx_hbm.at[pl.ds(base, bpw)], idx_v)
        pltpu.async_copy(emb_hbm.at[idx_v], rows_v, sem).wait()
        pltpu.sync_copy(rows_v, out_hbm.at[pl.ds(base, bpw)])

    return gather_kernel(emb, idx)


_NBUF = 3
_TILE_B = 32


def _tc_project(fe, wt, b2):
    B, K = fe.shape
    V = wt.shape[1]
    nb = B // _TILE_B

    def body(fe_ref, wt_ref, b_ref, out_hbm, out_bufs, out_sems):
        def out_copy(j):
            return pltpu.make_async_copy(
                out_bufs.at[j % _NBUF],
                out_hbm.at[pl.ds(j * _TILE_B, _TILE_B), :],
                out_sems.at[j % _NBUF],
            )

        bias = b_ref[...]
        for j in range(nb):
            if j >= _NBUF:
                out_copy(j - _NBUF).wait()
            out_bufs[j % _NBUF] = (
                lax.dot_general(
                    fe_ref[pl.ds(j * _TILE_B, _TILE_B), :], wt_ref[...],
                    dimension_numbers=(((1,), (0,)), ((), ())),
                    preferred_element_type=jnp.float32,
                )
                + bias
            )
            out_copy(j).start()
        for j in range(max(0, nb - _NBUF), nb):
            out_copy(j).wait()

    return pl.pallas_call(
        body,
        in_specs=[
            pl.BlockSpec(memory_space=pltpu.VMEM),
            pl.BlockSpec(memory_space=pltpu.VMEM),
            pl.BlockSpec(memory_space=pltpu.VMEM),
        ],
        out_specs=pl.BlockSpec(memory_space=pl.ANY),
        out_shape=jax.ShapeDtypeStruct((B, V), jnp.float32),
        compiler_params=pltpu.CompilerParams(needs_layout_passes=True),
        scratch_shapes=[
            pltpu.VMEM((_NBUF, _TILE_B, V), jnp.float32),
            pltpu.SemaphoreType.DMA((_NBUF,)),
        ],
    )(fe, wt, b2)


def kernel(x, emb, W, b):
    idx = x[:, -1].astype(jnp.int32)
    fe = _sc_gather(emb, idx)
    wt = W.T
    b2 = b.reshape(1, -1)
    return _tc_project(fe, wt, b2)


# speedup vs baseline: 2.7620x; 2.7620x over previous
import functools

import jax
import jax.numpy as jnp
from jax import lax
from jax.experimental import pallas as pl
from jax.experimental.pallas import tpu as pltpu
from jax.experimental.pallas import tpu_sc as plsc

_NC = 2
_NS = 16
_NW = _NC * _NS

_TILE_V = 2048


def _sc_gather(emb, idx):
    B = idx.shape[0]
    D = emb.shape[1]
    bpw = B // _NW
    mesh = plsc.VectorSubcoreMesh(
        core_axis_name="c", subcore_axis_name="s",
        num_cores=_NC, num_subcores=_NS,
    )

    @functools.partial(
        pl.kernel,
        out_type=jax.ShapeDtypeStruct((B, D), jnp.float32),
        mesh=mesh,
        compiler_params=pltpu.CompilerParams(use_tc_tiling_on_sc=False),
        scratch_types=[
            pltpu.VMEM((bpw,), jnp.int32),
            pltpu.VMEM((bpw, D), jnp.float32),
            pltpu.SemaphoreType.DMA,
        ],
    )
    def gather_kernel(emb_hbm, idx_hbm, out_hbm, idx_v, rows_v, sem):
        wid = lax.axis_index("s") * _NC + lax.axis_index("c")
        base = wid * bpw
        pltpu.sync_copy(idx_hbm.at[pl.ds(base, bpw)], idx_v)
        pltpu.async_copy(emb_hbm.at[idx_v], rows_v, sem).wait()
        pltpu.sync_copy(rows_v, out_hbm.at[pl.ds(base, bpw)])

    return gather_kernel(emb, idx)


def _tc_project_t(fe_aug, wt_aug):
    K, B = fe_aug.shape
    V = wt_aug.shape[1]
    nv = pl.cdiv(V, _TILE_V)

    def body(fe_ref, w_ref, out_ref):
        out_ref[...] = lax.dot_general(
            w_ref[...], fe_ref[...],
            dimension_numbers=(((0,), (0,)), ((), ())),
            preferred_element_type=jnp.float32,
        )

    return pl.pallas_call(
        body,
        grid=(nv,),
        in_specs=[
            pl.BlockSpec((K, B), lambda i: (0, 0)),
            pl.BlockSpec((K, _TILE_V), lambda i: (0, i)),
        ],
        out_specs=pl.BlockSpec((_TILE_V, B), lambda i: (i, 0)),
        out_shape=jax.ShapeDtypeStruct((V, B), jnp.float32),
    )(fe_aug, wt_aug)


def kernel(x, emb, W, b):
    idx = x[:, -1].astype(jnp.int32)
    fe = _sc_gather(emb, idx)
    fe_aug = jnp.concatenate(
        [fe.T, jnp.ones((1, fe.shape[0]), jnp.float32)], axis=0)
    wt_aug = jnp.concatenate([W.T, b[None, :]], axis=0)
    out_t = _tc_project_t(fe_aug, wt_aug)
    return out_t.T
